# baseline (speedup 1.0000x reference)
import functools

import jax
import jax.numpy as jnp
from jax import lax
from jax.experimental import pallas as pl
from jax.experimental.pallas import tpu as pltpu
from jax.experimental.pallas import tpu_sc as plsc

BATCH = 4096
N_ITEMS = 200
NUM_WORKERS = 32
ROWS = BATCH // NUM_WORKERS
LANES = 16
NCH = 4
CR = ROWS // NCH


def _gather_body(idx_hbm, table_hbm, out_hbm, idx2d, vals2d, idx1d, vals1d,
                 *sems):
    isems = sems[0:NCH]
    gsems = sems[NCH:2 * NCH]
    wsem = sems[2 * NCH]
    wid = lax.axis_index("s") * 2 + lax.axis_index("c")
    base = wid * ROWS

    def in_copy(c):
        return pltpu.make_async_copy(
            idx_hbm.at[pl.ds(base + c * CR, CR)],
            idx2d.at[pl.ds(c * CR, CR)],
            isems[c],
        )

    def gather_copy(c):
        sl = pl.ds(c * CR * N_ITEMS, CR * N_ITEMS)
        return pltpu.make_async_copy(
            table_hbm.at[idx1d.at[sl]], vals1d.at[sl], gsems[c]
        )

    def out_copy(c):
        return pltpu.make_async_copy(
            vals2d.at[pl.ds(c * CR, CR)],
            out_hbm.at[pl.ds(base + c * CR, CR)],
            wsem,
        )

    def detile_row(r, carry):
        for col in range(0, N_ITEMS - LANES + 1, LANES):
            idx1d[pl.ds(r * N_ITEMS + col, LANES)] = idx2d[r, pl.ds(col, LANES)]
        idx1d[pl.ds(r * N_ITEMS + N_ITEMS - LANES, LANES)] = idx2d[
            r, pl.ds(N_ITEMS - LANES, LANES)]
        return carry

    def retile_row(r, carry):
        vals2d[r, pl.ds(N_ITEMS - LANES, LANES)] = vals1d[
            pl.ds(r * N_ITEMS + N_ITEMS - LANES, LANES)]
        for col in range(0, N_ITEMS - LANES + 1, LANES):
            vals2d[r, pl.ds(col, LANES)] = vals1d[pl.ds(r * N_ITEMS + col, LANES)]
        return carry

    for c in range(NCH):
        in_copy(c).start()
    for c in range(NCH):
        in_copy(c).wait()
        lax.fori_loop(c * CR, (c + 1) * CR, detile_row, 0)
        gather_copy(c).start()
    for c in range(NCH):
        gather_copy(c).wait()
        lax.fori_loop(c * CR, (c + 1) * CR, retile_row, 0)
        out_copy(c).start()
    for c in range(NCH):
        out_copy(c).wait()


@jax.jit
def kernel(user_ids, item_ids, rank_table):
    del user_ids
    mesh = plsc.VectorSubcoreMesh(core_axis_name="c", subcore_axis_name="s")
    out = pl.kernel(
        _gather_body,
        out_type=jax.ShapeDtypeStruct((BATCH, N_ITEMS), jnp.float32),
        mesh=mesh,

        scratch_types=[
            pltpu.VMEM((ROWS, N_ITEMS), jnp.int32),
            pltpu.VMEM((ROWS, N_ITEMS), jnp.float32),
            pltpu.VMEM((ROWS * N_ITEMS,), jnp.int32),
            pltpu.VMEM((ROWS * N_ITEMS,), jnp.float32),
        ] + [pltpu.SemaphoreType.DMA] * (2 * NCH + 1),
    )(item_ids, rank_table)
    return out


# speedup vs baseline: 1.0231x; 1.0231x over previous
import functools

import jax
import jax.numpy as jnp
from jax import lax
from jax.experimental import pallas as pl
from jax.experimental.pallas import tpu as pltpu
from jax.experimental.pallas import tpu_sc as plsc

BATCH = 4096
N_ITEMS = 200
NUM_WORKERS = 32
ROWS = BATCH // NUM_WORKERS
LANES = 16
NCH = 8
CR = ROWS // NCH


def _gather_body(idx_hbm, table_hbm, out_hbm, idx2d, vals2d, idx1d, vals1d,
                 *sems):
    isems = sems[0:NCH]
    gsems = sems[NCH:2 * NCH]
    wsem = sems[2 * NCH]
    wid = lax.axis_index("s") * 2 + lax.axis_index("c")
    base = wid * ROWS

    def in_copy(c):
        return pltpu.make_async_copy(
            idx_hbm.at[pl.ds(base + c * CR, CR)],
            idx2d.at[pl.ds(c * CR, CR)],
            isems[c],
        )

    def gather_copy(c):
        sl = pl.ds(c * CR * N_ITEMS, CR * N_ITEMS)
        return pltpu.make_async_copy(
            table_hbm.at[idx1d.at[sl]], vals1d.at[sl], gsems[c]
        )

    def out_copy(c):
        return pltpu.make_async_copy(
            vals2d.at[pl.ds(c * CR, CR)],
            out_hbm.at[pl.ds(base + c * CR, CR)],
            wsem,
        )

    def detile_row(r, carry):
        for col in range(0, N_ITEMS - LANES + 1, LANES):
            idx1d[pl.ds(r * N_ITEMS + col, LANES)] = idx2d[r, pl.ds(col, LANES)]
        idx1d[pl.ds(r * N_ITEMS + N_ITEMS - LANES, LANES)] = idx2d[
            r, pl.ds(N_ITEMS - LANES, LANES)]
        return carry

    def retile_row(r, carry):
        vals2d[r, pl.ds(N_ITEMS - LANES, LANES)] = vals1d[
            pl.ds(r * N_ITEMS + N_ITEMS - LANES, LANES)]
        for col in range(0, N_ITEMS - LANES + 1, LANES):
            vals2d[r, pl.ds(col, LANES)] = vals1d[pl.ds(r * N_ITEMS + col, LANES)]
        return carry

    for c in range(NCH):
        in_copy(c).start()
    for c in range(NCH):
        in_copy(c).wait()
        lax.fori_loop(c * CR, (c + 1) * CR, detile_row, 0)
        gather_copy(c).start()
    for c in range(NCH):
        gather_copy(c).wait()
        lax.fori_loop(c * CR, (c + 1) * CR, retile_row, 0)
        out_copy(c).start()
    for c in range(NCH):
        out_copy(c).wait()


@jax.jit
def kernel(user_ids, item_ids, rank_table):
    del user_ids
    mesh = plsc.VectorSubcoreMesh(core_axis_name="c", subcore_axis_name="s")
    out = pl.kernel(
        _gather_body,
        out_type=jax.ShapeDtypeStruct((BATCH, N_ITEMS), jnp.float32),
        mesh=mesh,

        scratch_types=[
            pltpu.VMEM((ROWS, N_ITEMS), jnp.int32),
            pltpu.VMEM((ROWS, N_ITEMS), jnp.float32),
            pltpu.VMEM((ROWS * N_ITEMS,), jnp.int32),
            pltpu.VMEM((ROWS * N_ITEMS,), jnp.float32),
        ] + [pltpu.SemaphoreType.DMA] * (2 * NCH + 1),
    )(item_ids, rank_table)
    return out
